# Initial kernel scaffold; baseline (speedup 1.0000x reference)
import jax
import jax.numpy as jnp
from jax.experimental import pallas as pl


def kernel(x, table):
    raise NotImplementedError("write your pallas kernel here")


# speedup vs baseline: 94.5712x; 94.5712x over previous
import jax
import jax.numpy as jnp
from jax.experimental import pallas as pl
from jax.experimental.pallas import tpu as pltpu

_NH = 16
_NB = 32
_Q = 2048
_K = 2048
_LINE = 2 * _Q
_THR = (8, 12, 16, 23, 32, 46, 64, 91)
_NA = 16
_NR = 128


def _line_kernel(table_ref, line_ref):
    j = jax.lax.broadcasted_iota(jnp.int32, (1, _LINE), 1)
    d = j - (_Q - 1)
    a = jnp.abs(d)
    v = jnp.minimum(a, 7)
    for t in _THR:
        v = v + (a >= t).astype(jnp.int32)
    bucket = jnp.where(d > 0, 16, 0) + v
    acc = jnp.zeros((_NH, _LINE), jnp.float32)
    for b in range(_NB):
        col = table_ref[b, :].reshape(_NH, 1)
        acc = jnp.where(bucket == b, col, acc)
    line_ref[...] = acc


def _copies(u_ref, out_ref, sems, par, r):
    cps = []
    for a in range(_NA):
        cps.append(pltpu.make_async_copy(
            u_ref.at[par, :, pl.ds((_NA - 1 - a) * 128, _K)],
            out_ref.at[:, a * _NR + r, :],
            sems.at[par, a],
        ))
    return cps


def _expand_kernel(line_ref, out_ref, u_ref, sems):
    i = pl.program_id(0)
    par = jax.lax.rem(i, 2)

    shift = jax.lax.rem(jnp.int32(_LINE - 127) + i, jnp.int32(_LINE))
    u_ref[par] = pltpu.roll(line_ref[...], shift, 1)

    @pl.when(i >= 1)
    def _():
        for cp in _copies(u_ref, out_ref, sems, 1 - par, i - 1):
            cp.wait()

    for cp in _copies(u_ref, out_ref, sems, par, i):
        cp.start()

    @pl.when(i == _NR - 1)
    def _():
        for cp in _copies(u_ref, out_ref, sems, par, i):
            cp.wait()


def kernel(x, table):
    del x
    line = pl.pallas_call(
        _line_kernel,
        out_shape=jax.ShapeDtypeStruct((_NH, _LINE), jnp.float32),
    )(table)
    return pl.pallas_call(
        _expand_kernel,
        grid=(_NR,),
        in_specs=[pl.BlockSpec((_NH, _LINE), lambda r: (0, 0))],
        out_specs=pl.BlockSpec(memory_space=pl.ANY),
        out_shape=jax.ShapeDtypeStruct((_NH, _Q, _K), jnp.float32),
        scratch_shapes=[
            pltpu.VMEM((2, _NH, _LINE), jnp.float32),
            pltpu.SemaphoreType.DMA((2, _NA)),
        ],
    )(line)


# speedup vs baseline: 189.1794x; 2.0004x over previous
import jax
import jax.numpy as jnp
from jax.experimental import pallas as pl
from jax.experimental.pallas import tpu as pltpu

_NH = 16
_NB = 32
_Q = 2048
_K = 2048
_LINE = 2 * _Q
_THR = (8, 12, 16, 23, 32, 46, 64, 91)
_NA = 16
_NR = 128


def _line_kernel(table_ref, line_ref):
    j = jax.lax.broadcasted_iota(jnp.int32, (1, _LINE), 1)
    d = j - (_Q - 1)
    a = jnp.abs(d)
    v = jnp.minimum(a, 7)
    for t in _THR:
        v = v + (a >= t).astype(jnp.int32)
    bucket = jnp.where(d > 0, 16, 0) + v
    acc = jnp.zeros((_NH, _LINE), jnp.float32)
    for b in range(_NB):
        col = table_ref[b, :].reshape(_NH, 1)
        acc = jnp.where(bucket == b, col, acc)
    line_ref[...] = acc


def _copies(u_ref, out_ref, sems, par, r):
    cps = []
    for a in range(_NA):
        cps.append(pltpu.make_async_copy(
            u_ref.at[par, :, pl.ds((_NA - 1 - a) * 128, _K)],
            out_ref.at[:, a * _NR + r, :],
            sems.at[par, a],
        ))
    return cps


_NBUF = 4


def _expand_kernel(line_ref, out_ref, u_ref, sems):
    i = pl.program_id(0)
    par = jax.lax.rem(i, _NBUF)

    @pl.when(i >= _NBUF)
    def _():
        for cp in _copies(u_ref, out_ref, sems, par, i - _NBUF):
            cp.wait()

    shift = jax.lax.rem(jnp.int32(_LINE - 127) + i, jnp.int32(_LINE))
    u_ref[par] = pltpu.roll(line_ref[...], shift, 1)

    for cp in _copies(u_ref, out_ref, sems, par, i):
        cp.start()

    @pl.when(i == _NR - 1)
    def _():
        for back in range(_NBUF - 1, -1, -1):
            s = i - back
            p = jax.lax.rem(jnp.int32(s), _NBUF)
            for cp in _copies(u_ref, out_ref, sems, p, s):
                cp.wait()


def kernel(x, table):
    del x
    line = pl.pallas_call(
        _line_kernel,
        out_shape=jax.ShapeDtypeStruct((_NH, _LINE), jnp.float32),
    )(table)
    return pl.pallas_call(
        _expand_kernel,
        grid=(_NR,),
        in_specs=[pl.BlockSpec((_NH, _LINE), lambda r: (0, 0))],
        out_specs=pl.BlockSpec(memory_space=pl.ANY),
        out_shape=jax.ShapeDtypeStruct((_NH, _Q, _K), jnp.float32),
        scratch_shapes=[
            pltpu.VMEM((_NBUF, _NH, _LINE), jnp.float32),
            pltpu.SemaphoreType.DMA((_NBUF, _NA)),
        ],
    )(line)
